# Initial kernel scaffold; baseline (speedup 1.0000x reference)
import jax
import jax.numpy as jnp
from jax.experimental import pallas as pl


def kernel(g, features, W1, b1, W2, b2, W3, b3):
    raise NotImplementedError("write your pallas kernel here")


# speedup vs baseline: 9.5265x; 9.5265x over previous
import functools

import jax
import jax.numpy as jnp
from jax import lax
from jax.experimental import pallas as pl
from jax.experimental.pallas import tpu as pltpu
from jax.experimental.pallas import tpu_sc as plsc

N = 10000
E = 320000
D = 128
NCLS = 40
NC = 2
NS = 16
NW = NC * NS
EB = E // 128
CHUNKS = -(-EB // NW)
RPW = 624
TAIL = N - NS * RPW


def _mesh():
    return plsc.VectorSubcoreMesh(core_axis_name="c", subcore_axis_name="s")


def _sc_degree(dst_blocks):
    @functools.partial(
        pl.kernel,
        mesh=_mesh(),
        out_type=jax.ShapeDtypeStruct((NC * N, 128), jnp.float32),
        scratch_types=[
            pltpu.VMEM((128,), jnp.int32),
            pltpu.VMEM((128, 128), jnp.float32),
            pltpu.VMEM((128, 128), jnp.float32),
            pltpu.VMEM_SHARED((N, 128), jnp.float32),
        ],
    )
    def k(dst_hbm, out_hbm, dst_v, ones_v, zeros_v, deg_sh):
        c = lax.axis_index("c")
        s = lax.axis_index("s")
        wid = c * NS + s
        zeros16 = jnp.zeros((16,), jnp.float32)
        ones16 = jnp.ones((16,), jnp.float32)

        def fill(r, carry):
            for j in range(8):
                ones_v[r, pl.ds(j * 16, 16)] = ones16
                zeros_v[r, pl.ds(j * 16, 16)] = zeros16
            return carry

        lax.fori_loop(0, 128, fill, 0)

        base = s * RPW
        for kk in range(4):
            pltpu.sync_copy(zeros_v, deg_sh.at[pl.ds(base + kk * 128, 128)])
        pltpu.sync_copy(zeros_v.at[pl.ds(0, RPW - 512)],
                        deg_sh.at[pl.ds(base + 512, RPW - 512)])

        @pl.when(s == 0)
        def _():
            pltpu.sync_copy(zeros_v.at[pl.ds(0, TAIL)],
                            deg_sh.at[pl.ds(NS * RPW, TAIL)])

        plsc.subcore_barrier()

        def body(i, carry):
            cid = wid + NW * i

            @pl.when(cid < EB)
            def _():
                pltpu.sync_copy(dst_hbm.at[cid], dst_v)
                pltpu.sync_copy(ones_v, deg_sh.at[dst_v], add=True)

            return carry

        lax.fori_loop(0, CHUNKS, body, 0)
        plsc.subcore_barrier()
        pltpu.sync_copy(deg_sh.at[pl.ds(base, RPW)],
                        out_hbm.at[pl.ds(c * N + base, RPW)])

        @pl.when(s == 0)
        def _():
            pltpu.sync_copy(deg_sh.at[pl.ds(NS * RPW, TAIL)],
                            out_hbm.at[pl.ds(c * N + NS * RPW, TAIL)])

    return k(dst_blocks)


def _sc_gather_scatter(hn, src_blocks, dst_blocks):
    @functools.partial(
        pl.kernel,
        mesh=_mesh(),
        out_type=jax.ShapeDtypeStruct((NC * N, D), jnp.float32),
        scratch_types=[
            pltpu.VMEM((128,), jnp.int32),
            pltpu.VMEM((128,), jnp.int32),
            pltpu.VMEM((128, D), jnp.float32),
            pltpu.VMEM_SHARED((N, D), jnp.float32),
            pltpu.SemaphoreType.DMA,
        ],
    )
    def k(hn_hbm, src_hbm, dst_hbm, out_hbm, src_v, dst_v, rows_v, agg_sh, sem):
        c = lax.axis_index("c")
        s = lax.axis_index("s")
        wid = c * NS + s
        zeros16 = jnp.zeros((16,), jnp.float32)

        def zb(r, carry):
            for j in range(8):
                rows_v[r, pl.ds(j * 16, 16)] = zeros16
            return carry

        lax.fori_loop(0, 128, zb, 0)
        base = s * RPW
        for kk in range(4):
            pltpu.sync_copy(rows_v, agg_sh.at[pl.ds(base + kk * 128, 128)])
        pltpu.sync_copy(rows_v.at[pl.ds(0, RPW - 512)],
                        agg_sh.at[pl.ds(base + 512, RPW - 512)])

        @pl.when(s == 0)
        def _():
            pltpu.sync_copy(rows_v.at[pl.ds(0, TAIL)],
                            agg_sh.at[pl.ds(NS * RPW, TAIL)])

        plsc.subcore_barrier()

        def body(i, carry):
            cid = wid + NW * i

            @pl.when(cid < EB)
            def _():
                pltpu.sync_copy(src_hbm.at[cid], src_v)
                pltpu.sync_copy(dst_hbm.at[cid], dst_v)
                pltpu.async_copy(hn_hbm.at[src_v], rows_v, sem).wait()
                pltpu.sync_copy(rows_v, agg_sh.at[dst_v], add=True)

            return carry

        lax.fori_loop(0, CHUNKS, body, 0)
        plsc.subcore_barrier()
        pltpu.sync_copy(agg_sh.at[pl.ds(base, RPW)],
                        out_hbm.at[pl.ds(c * N + base, RPW)])

        @pl.when(s == 0)
        def _():
            pltpu.sync_copy(agg_sh.at[pl.ds(NS * RPW, TAIL)],
                            out_hbm.at[pl.ds(c * N + NS * RPW, TAIL)])

    return k(hn, src_blocks, dst_blocks)


_R = 1000


def _tc_first(degp, features, W1):
    def body(degp_ref, f_ref, w_ref, norm_ref, hn_ref):
        deg2 = degp_ref[0, :, 0:1] + degp_ref[1, :, 0:1]
        nrm2 = lax.rsqrt(jnp.maximum(deg2, 1.0))
        norm_ref[...] = nrm2
        h = jnp.dot(f_ref[...], w_ref[...], preferred_element_type=jnp.float32)
        hn_ref[...] = h * nrm2

    return pl.pallas_call(
        body,
        grid=(N // _R,),
        in_specs=[
            pl.BlockSpec((NC, _R, 128), lambda i: (0, i, 0)),
            pl.BlockSpec((_R, D), lambda i: (i, 0)),
            pl.BlockSpec((D, D), lambda i: (0, 0)),
        ],
        out_specs=[
            pl.BlockSpec((_R, 1), lambda i: (i, 0)),
            pl.BlockSpec((_R, D), lambda i: (i, 0)),
        ],
        out_shape=[
            jax.ShapeDtypeStruct((N, 1), jnp.float32),
            jax.ShapeDtypeStruct((N, D), jnp.float32),
        ],
    )(degp, features, W1)


def _tc_mid(pflat, norm, b, W):
    def body(p0_ref, p1_ref, n_ref, b_ref, w_ref, out_ref):
        agg = p0_ref[...] + p1_ref[...]
        h = jnp.maximum(agg * n_ref[...] + b_ref[...][None, :], 0.0)
        out_ref[...] = jnp.dot(h, w_ref[...],
                               preferred_element_type=jnp.float32) * n_ref[...]

    return pl.pallas_call(
        body,
        grid=(N // _R,),
        in_specs=[
            pl.BlockSpec((_R, D), lambda i: (i, 0)),
            pl.BlockSpec((_R, D), lambda i: (i + N // _R, 0)),
            pl.BlockSpec((_R, 1), lambda i: (i, 0)),
            pl.BlockSpec((D,), lambda i: (0,)),
            pl.BlockSpec((D, D), lambda i: (0, 0)),
        ],
        out_specs=pl.BlockSpec((_R, D), lambda i: (i, 0)),
        out_shape=jax.ShapeDtypeStruct((N, D), jnp.float32),
    )(pflat, pflat, norm, b, W)


def _tc_final(qflat, norm, b, W3, b3):
    def body(q0_ref, q1_ref, n_ref, b_ref, w3_ref, b3_ref, out_ref):
        agg = q0_ref[...] + q1_ref[...]
        h = jnp.maximum(agg * n_ref[...] + b_ref[...][None, :], 0.0)
        out_ref[...] = jnp.dot(h, w3_ref[...],
                               preferred_element_type=jnp.float32) + b3_ref[...][None, :]

    return pl.pallas_call(
        body,
        grid=(N // _R,),
        in_specs=[
            pl.BlockSpec((_R, D), lambda i: (i, 0)),
            pl.BlockSpec((_R, D), lambda i: (i + N // _R, 0)),
            pl.BlockSpec((_R, 1), lambda i: (i, 0)),
            pl.BlockSpec((D,), lambda i: (0,)),
            pl.BlockSpec((D, NCLS), lambda i: (0, 0)),
            pl.BlockSpec((NCLS,), lambda i: (0,)),
        ],
        out_specs=pl.BlockSpec((_R, NCLS), lambda i: (i, 0)),
        out_shape=jax.ShapeDtypeStruct((N, NCLS), jnp.float32),
    )(qflat, qflat, norm, b, W3, b3)


def kernel(g, features, W1, b1, W2, b2, W3, b3):
    src_b = g[0].reshape(EB, 128)
    dst_b = g[1].reshape(EB, 128)
    degp = _sc_degree(dst_b).reshape(NC, N, 128)
    norm, hn1 = _tc_first(degp, features, W1)
    p = _sc_gather_scatter(hn1, src_b, dst_b)
    hn2 = _tc_mid(p, norm, b1, W2)
    q = _sc_gather_scatter(hn2, src_b, dst_b)
    return _tc_final(q, norm, b2, W3, b3)
